# Initial kernel scaffold; baseline (speedup 1.0000x reference)
import jax
import jax.numpy as jnp
from jax.experimental import pallas as pl


def kernel(token_ids, weights):
    raise NotImplementedError("write your pallas kernel here")


# speedup vs baseline: 1.8445x; 1.8445x over previous
import functools

import jax
import jax.numpy as jnp
from jax import lax
from jax.experimental import pallas as pl
from jax.experimental.pallas import tpu as pltpu
from jax.experimental.pallas import tpu_sc as plsc

_NUM_WORKERS = 32
_CHUNK = 1024
_GATHER = 128


@functools.partial(jax.jit, static_argnums=(2, 3))
def _emb_lookup(flat_ids, weights, b_total, d_model):
  rows_per_w = b_total // _NUM_WORKERS
  n_chunks = rows_per_w // _CHUNK
  mesh = plsc.VectorSubcoreMesh(core_axis_name="c", subcore_axis_name="s")

  @functools.partial(
      pl.kernel,
      mesh=mesh,
      out_type=jax.ShapeDtypeStruct((b_total, d_model), jnp.float32),
      scratch_types=[
          pltpu.VMEM((_CHUNK,), jnp.int32),
          pltpu.VMEM((_CHUNK, d_model), jnp.float32),
          pltpu.SemaphoreType.DMA,
      ],
      compiler_params=pltpu.CompilerParams(use_tc_tiling_on_sc=False),
  )
  def emb_kernel(idx_hbm, table_hbm, out_hbm, idx_v, rows_v, sem):
    wid = lax.axis_index("s") * 2 + lax.axis_index("c")
    base = wid * rows_per_w

    def chunk_body(j, carry):
      cbase = base + j * _CHUNK
      pltpu.sync_copy(idx_hbm.at[pl.ds(cbase, _CHUNK)], idx_v)
      copies = []
      for g in range(_CHUNK // _GATHER):
        copies.append(
            pltpu.async_copy(
                table_hbm.at[idx_v.at[pl.ds(g * _GATHER, _GATHER)]],
                rows_v.at[pl.ds(g * _GATHER, _GATHER)],
                sem,
            ))
      for c in copies:
        c.wait()
      pltpu.sync_copy(rows_v, out_hbm.at[pl.ds(cbase, _CHUNK)])
      return carry

    lax.fori_loop(0, n_chunks, chunk_body, 0)

  return emb_kernel(flat_ids, weights)


def kernel(token_ids, weights):
  bsz, seq = token_ids.shape
  d_model = weights.shape[1]
  flat = token_ids.reshape(-1).astype(jnp.int32)
  out = _emb_lookup(flat, weights, bsz * seq, d_model)
  return out.reshape(bsz, seq, d_model)


# speedup vs baseline: 1.8685x; 1.0130x over previous
import functools

import jax
import jax.numpy as jnp
from jax import lax
from jax.experimental import pallas as pl
from jax.experimental.pallas import tpu as pltpu
from jax.experimental.pallas import tpu_sc as plsc

_NUM_WORKERS = 32
_NBUF = 8
_CHUNK = 200
_GSIZES = (128, 72)


@functools.partial(jax.jit, static_argnums=(2, 3))
def _emb_lookup(flat_ids, weights, b_total, d_model):
  rows_per_w = b_total // _NUM_WORKERS
  n_chunks = rows_per_w // _CHUNK
  n_rounds = n_chunks // _NBUF
  assert rows_per_w == n_rounds * _NBUF * _CHUNK
  mesh = plsc.VectorSubcoreMesh(core_axis_name="c", subcore_axis_name="s")

  @functools.partial(
      pl.kernel,
      mesh=mesh,
      out_type=jax.ShapeDtypeStruct((b_total, d_model), jnp.float32),
      scratch_types=(
          [pltpu.VMEM((_NBUF * _CHUNK,), jnp.int32),
           pltpu.VMEM((_NBUF * _CHUNK, d_model), jnp.float32)]
          + [pltpu.SemaphoreType.DMA] * (3 * _NBUF)
      ),
      compiler_params=pltpu.CompilerParams(use_tc_tiling_on_sc=False),
  )
  def emb_kernel(idx_hbm, table_hbm, out_hbm, idx_v, rows_v, *sems):
    sem_i = sems[0:_NBUF]
    sem_g = sems[_NBUF:2 * _NBUF]
    sem_o = sems[2 * _NBUF:3 * _NBUF]
    wid = lax.axis_index("s") * 2 + lax.axis_index("c")
    base = wid * rows_per_w

    def idx_desc(r, b):
      j = r * _NBUF + b
      return pltpu.make_async_copy(
          idx_hbm.at[pl.ds(base + j * _CHUNK, _CHUNK)],
          idx_v.at[pl.ds(b * _CHUNK, _CHUNK)],
          sem_i[b])

    def out_desc(r, b):
      j = r * _NBUF + b
      return pltpu.make_async_copy(
          rows_v.at[pl.ds(b * _CHUNK, _CHUNK)],
          out_hbm.at[pl.ds(base + j * _CHUNK, _CHUNK)],
          sem_o[b])

    for b in range(_NBUF):
      idx_desc(0, b).start()

    @pl.loop(0, n_rounds)
    def round_body(r):
      for b in range(_NBUF):
        @pl.when(r > 0)
        def _(b=b):
          out_desc(r - 1, b).wait()
      gd = []
      for b in range(_NBUF):
        idx_desc(r, b).wait()
        off = 0
        for gs in _GSIZES:
          gd.append(pltpu.async_copy(
              table_hbm.at[idx_v.at[pl.ds(b * _CHUNK + off, gs)]],
              rows_v.at[pl.ds(b * _CHUNK + off, gs)],
              sem_g[b]))
          off += gs
      k = 0
      for b in range(_NBUF):
        for _gs in _GSIZES:
          gd[k].wait()
          k += 1
        out_desc(r, b).start()
        @pl.when(r < n_rounds - 1)
        def _(b=b):
          idx_desc(r + 1, b).start()

    for b in range(_NBUF):
      out_desc(n_rounds - 1, b).wait()

  return emb_kernel(flat_ids, weights)


def kernel(token_ids, weights):
  bsz, seq = token_ids.shape
  d_model = weights.shape[1]
  flat = token_ids.reshape(-1).astype(jnp.int32)
  out = _emb_lookup(flat, weights, bsz * seq, d_model)
  return out.reshape(bsz, seq, d_model)
